# baseline (speedup 1.0000x reference)
import functools

import jax
import jax.numpy as jnp
from jax import lax
from jax.experimental import pallas as pl
from jax.experimental.pallas import tpu as pltpu, tpu_sc as plsc

OUT = 7
ROIS = 512
ROI_BLOCK = 64
C = 256
LVL_H = (200, 100, 50, 25)
LVL_BASE = (0, 80000, 100000, 105000)
LVL_BHW = (40000, 10000, 2500, 625)
TABLE_ROWS = 106250

NC, NS = 2, 16
NW = NC * NS
ROIS_PER_W = ROIS // NW


def _sel(lvl, vals, dtype):
    out = jnp.full(lvl.shape, vals[3], dtype=dtype)
    for l in (2, 1, 0):
        out = jnp.where(lvl == l, jnp.asarray(vals[l], dtype), out)
    return out


def _prologue_body(raw_ref, idx_ref, w_ref):
    raw = raw_ref[...]
    i = pl.program_id(0)
    rid = i * ROI_BLOCK + lax.broadcasted_iota(jnp.int32, (ROI_BLOCK, 1), 0)
    b = rid // 256

    x1 = raw[:, 0:1] * 600.0
    y1 = raw[:, 1:2] * 600.0
    x2 = x1 + 16.0 + raw[:, 2:3] * 200.0
    y2 = y1 + 16.0 + raw[:, 3:4] * 200.0
    area = (x2 - x1) * (y2 - y1)
    size = jnp.sqrt(area)
    lvlf = jnp.floor(4.0 + jnp.log2(size / 224.0 + 1e-8))
    lvl = jnp.clip(lvlf, 2.0, 5.0).astype(jnp.int32) - 2

    scale = _sel(lvl, (0.25, 0.125, 0.0625, 0.03125), jnp.float32)
    Hs = _sel(lvl, LVL_H, jnp.int32)
    Ws = Hs
    base = _sel(lvl, LVL_BASE, jnp.int32) + b * _sel(lvl, LVL_BHW, jnp.int32)
    Hf = Hs.astype(jnp.float32)
    Wf = Hf

    x1s = x1 * scale - 0.5
    y1s = y1 * scale - 0.5
    x2s = x2 * scale - 0.5
    y2s = y2 * scale - 0.5
    bin_w = (x2s - x1s) / OUT
    bin_h = (y2s - y1s) / OUT

    J = lax.broadcasted_iota(jnp.int32, (1, 784), 1)
    syi = J // 56
    sxi = (J % 56) // 4
    cr = J % 4
    gy = (syi // 2).astype(jnp.float32) + ((syi % 2).astype(jnp.float32) + 0.5) / 2.0
    gx = (sxi // 2).astype(jnp.float32) + ((sxi % 2).astype(jnp.float32) + 0.5) / 2.0

    sy = y1s + gy * bin_h
    sx = x1s + gx * bin_w
    vy = (sy > -1.0) & (sy < Hf)
    vx = (sx > -1.0) & (sx < Wf)
    yc = jnp.clip(sy, 0.0, Hf - 1.0)
    xc = jnp.clip(sx, 0.0, Wf - 1.0)
    y0f = jnp.floor(yc)
    x0f = jnp.floor(xc)
    y0i = y0f.astype(jnp.int32)
    x0i = x0f.astype(jnp.int32)
    y1i = jnp.minimum(y0i + 1, Hs - 1)
    x1i = jnp.minimum(x0i + 1, Ws - 1)
    ly = yc - y0f
    lx = xc - x0f
    hy = 1.0 - ly
    hx = 1.0 - lx

    yhi = cr >= 2
    xhi = (cr % 2) == 1
    yidx = jnp.where(yhi, y1i, y0i)
    wy = jnp.where(yhi, ly, hy)
    xidx = jnp.where(xhi, x1i, x0i)
    wx = jnp.where(xhi, lx, hx)

    idx = base + yidx * Ws + xidx
    wgt = jnp.where(vy & vx, wy * wx * 0.25, 0.0)
    idx_ref[...] = idx
    w_ref[...] = wgt


def _run_prologue(raw):
    grid = ROIS // ROI_BLOCK
    return pl.pallas_call(
        _prologue_body,
        grid=(grid,),
        in_specs=[pl.BlockSpec((ROI_BLOCK, 4), lambda i: (i, 0))],
        out_specs=[
            pl.BlockSpec((ROI_BLOCK, 784), lambda i: (i, 0)),
            pl.BlockSpec((ROI_BLOCK, 784), lambda i: (i, 0)),
        ],
        out_shape=[
            jax.ShapeDtypeStruct((ROIS, 784), jnp.int32),
            jax.ShapeDtypeStruct((ROIS, 784), jnp.float32),
        ],
    )(raw)


CHUNKS = ROIS_PER_W * OUT
CW = C // 2


def _sc_body(table, idxs, ws, out,
             idx0, idx1, w0, w1, rows0, rows1, out_v, sem0, sem1):
    idx_v = (idx0, idx1)
    w_v = (w0, w1)
    rows_v = (rows0, rows1)
    sems = (sem0, sem1)
    wid = lax.axis_index("s") * NC + lax.axis_index("c")
    roi0 = wid * ROIS_PER_W
    shift16 = jnp.full((16,), 16, jnp.int32)
    mask_hi = jnp.full((16,), -65536, jnp.int32)
    even16 = lax.iota(jnp.int32, 16) * 2

    def stage(t, b):
        n = roi0 + t // OUT
        ph = t % OUT
        pltpu.sync_copy(idxs.at[n, ph], idx_v[b])
        pltpu.sync_copy(ws.at[n, ph], w_v[b])
        pltpu.async_copy(table.at[idx_v[b]], rows_v[b], sems[b])

    stage(jnp.int32(0), 0)

    def tt_body(tt, _):
        for b in (0, 1):
            t = tt * 2 + b
            nb = 1 - b

            @pl.when(t + 1 < CHUNKS)
            def _():
                stage(t + 1, nb)

            pltpu.make_async_copy(table.at[idx_v[b]], rows_v[b], sems[b]).wait()
            n = roi0 + t // OUT
            ph = t % OUT

            def pw_body(pw, _, b=b):
                accs = [jnp.zeros((16,), jnp.float32) for _ in range(16)]
                for iy in range(2):
                    for j in range(8):
                        kk = iy * 56 + pw * 8 + j
                        wv = w_v[b][kk]
                        for c in range(8):
                            xi = rows_v[b][kk, pl.ds(c * 16, 16)]
                            lo = lax.bitcast_convert_type(
                                lax.shift_left(xi, shift16), jnp.float32
                            )
                            hi = lax.bitcast_convert_type(
                                lax.bitwise_and(xi, mask_hi), jnp.float32
                            )
                            accs[2 * c] = accs[2 * c] + wv * lo
                            accs[2 * c + 1] = accs[2 * c + 1] + wv * hi
                for c in range(8):
                    out_v[pw, pl.ds(c * 32, 16)] = accs[2 * c]
                    out_v[pw, pl.ds(c * 32 + 16, 16)] = accs[2 * c + 1]
                return 0

            lax.fori_loop(0, OUT, pw_body, 0)
            pltpu.sync_copy(out_v, out.at[n, ph])
        return 0

    lax.fori_loop(0, CHUNKS // 2, tt_body, 0)


def _run_sc(table, idx, w):
    mesh = plsc.VectorSubcoreMesh(core_axis_name="c", subcore_axis_name="s")
    f = functools.partial(
        pl.kernel,
        out_type=jax.ShapeDtypeStruct((ROIS, OUT, OUT, C), jnp.float32),
        mesh=mesh,
        scratch_types=[
            pltpu.VMEM((112,), jnp.int32),
            pltpu.VMEM((112,), jnp.int32),
            pltpu.VMEM((112, 16), jnp.float32),
            pltpu.VMEM((112, 16), jnp.float32),
            pltpu.VMEM((112, CW), jnp.int32),
            pltpu.VMEM((112, CW), jnp.int32),
            pltpu.VMEM((OUT, C), jnp.float32),
            pltpu.SemaphoreType.DMA,
            pltpu.SemaphoreType.DMA,
        ],
    )(_sc_body)
    return f(table, idx, w)


_PERM = tuple(
    (p // 32) * 32 + (p % 2) * 16 + (p % 32) // 2 for p in range(C)
)


def kernel(feat_p2, feat_p3, feat_p4, feat_p5, boxes_raw):
    feats = (feat_p2, feat_p3, feat_p4, feat_p5)
    table_bf = jnp.concatenate(
        [
            f.transpose(0, 2, 3, 1).astype(jnp.bfloat16).reshape(-1, C)
            for f in feats
        ],
        axis=0,
    )
    table = lax.bitcast_convert_type(
        table_bf.reshape(TABLE_ROWS, CW, 2), jnp.int32
    )
    raw = boxes_raw.reshape(ROIS, 4)
    idx_flat, w_flat = _run_prologue(raw)
    idx = idx_flat.reshape(ROIS, OUT, 112)
    w = jnp.broadcast_to(
        w_flat.reshape(ROIS, OUT, 112, 1), (ROIS, OUT, 112, 16)
    )
    out = _run_sc(table, idx, w)
    return (
        out.reshape(ROIS, OUT, OUT, 8, 2, 16)
        .transpose(0, 3, 5, 4, 1, 2)
        .reshape(ROIS, C, OUT, OUT)
    )


# speedup vs baseline: 2.6393x; 2.6393x over previous
import functools

import jax
import jax.numpy as jnp
from jax import lax
from jax.experimental import pallas as pl
from jax.experimental.pallas import tpu as pltpu, tpu_sc as plsc

OUT = 7
ROIS = 512
ROI_BLOCK = 64
C = 256
LVL_H = (200, 100, 50, 25)
LVL_BASE = (0, 80000, 100000, 105000)
LVL_BHW = (40000, 10000, 2500, 625)
TABLE_ROWS = 106250

NC, NS = 2, 16
NW = NC * NS
ROIS_PER_W = ROIS // NW


def _sel(lvl, vals, dtype):
    out = jnp.full(lvl.shape, vals[3], dtype=dtype)
    for l in (2, 1, 0):
        out = jnp.where(lvl == l, jnp.asarray(vals[l], dtype), out)
    return out


def _prologue_body(raw_ref, idx_ref, w_ref):
    raw = raw_ref[...]
    i = pl.program_id(0)
    rid = i * ROI_BLOCK + lax.broadcasted_iota(jnp.int32, (ROI_BLOCK, 1), 0)
    b = rid // 256

    x1 = raw[:, 0:1] * 600.0
    y1 = raw[:, 1:2] * 600.0
    x2 = x1 + 16.0 + raw[:, 2:3] * 200.0
    y2 = y1 + 16.0 + raw[:, 3:4] * 200.0
    area = (x2 - x1) * (y2 - y1)
    size = jnp.sqrt(area)
    lvlf = jnp.floor(4.0 + jnp.log2(size / 224.0 + 1e-8))
    lvl = jnp.clip(lvlf, 2.0, 5.0).astype(jnp.int32) - 2

    scale = _sel(lvl, (0.25, 0.125, 0.0625, 0.03125), jnp.float32)
    Hs = _sel(lvl, LVL_H, jnp.int32)
    Ws = Hs
    base = _sel(lvl, LVL_BASE, jnp.int32) + b * _sel(lvl, LVL_BHW, jnp.int32)
    Hf = Hs.astype(jnp.float32)
    Wf = Hf

    x1s = x1 * scale - 0.5
    y1s = y1 * scale - 0.5
    x2s = x2 * scale - 0.5
    y2s = y2 * scale - 0.5
    bin_w = (x2s - x1s) / OUT
    bin_h = (y2s - y1s) / OUT

    J = lax.broadcasted_iota(jnp.int32, (1, 784), 1)
    syi = J // 56
    sxi = (J % 56) // 4
    cr = J % 4
    gy = (syi // 2).astype(jnp.float32) + ((syi % 2).astype(jnp.float32) + 0.5) / 2.0
    gx = (sxi // 2).astype(jnp.float32) + ((sxi % 2).astype(jnp.float32) + 0.5) / 2.0

    sy = y1s + gy * bin_h
    sx = x1s + gx * bin_w
    vy = (sy > -1.0) & (sy < Hf)
    vx = (sx > -1.0) & (sx < Wf)
    yc = jnp.clip(sy, 0.0, Hf - 1.0)
    xc = jnp.clip(sx, 0.0, Wf - 1.0)
    y0f = jnp.floor(yc)
    x0f = jnp.floor(xc)
    y0i = y0f.astype(jnp.int32)
    x0i = x0f.astype(jnp.int32)
    y1i = jnp.minimum(y0i + 1, Hs - 1)
    x1i = jnp.minimum(x0i + 1, Ws - 1)
    ly = yc - y0f
    lx = xc - x0f
    hy = 1.0 - ly
    hx = 1.0 - lx

    yhi = cr >= 2
    xhi = (cr % 2) == 1
    yidx = jnp.where(yhi, y1i, y0i)
    wy = jnp.where(yhi, ly, hy)
    xidx = jnp.where(xhi, x1i, x0i)
    wx = jnp.where(xhi, lx, hx)

    idx = base + yidx * Ws + xidx
    wgt = jnp.where(vy & vx, wy * wx * 0.25, 0.0)
    idx_ref[...] = idx
    w_ref[...] = wgt


def _run_prologue(raw):
    grid = ROIS // ROI_BLOCK
    return pl.pallas_call(
        _prologue_body,
        grid=(grid,),
        in_specs=[pl.BlockSpec((ROI_BLOCK, 4), lambda i: (i, 0))],
        out_specs=[
            pl.BlockSpec((ROI_BLOCK, 784), lambda i: (i, 0)),
            pl.BlockSpec((ROI_BLOCK, 784), lambda i: (i, 0)),
        ],
        out_shape=[
            jax.ShapeDtypeStruct((ROIS, 784), jnp.int32),
            jax.ShapeDtypeStruct((ROIS, 784), jnp.float32),
        ],
    )(raw)


CHUNKS = ROIS_PER_W * OUT
CW = C // 2


def _sc_body(table, idxs, ws, out,
             idx0, idx1, w0, w1, rows0, rows1, out_v, sem0, sem1):
    idx_v = (idx0, idx1)
    w_v = (w0, w1)
    rows_v = (rows0, rows1)
    sems = (sem0, sem1)
    wid = lax.axis_index("s") * NC + lax.axis_index("c")
    roi0 = wid * ROIS_PER_W
    shift16 = jnp.full((16,), 16, jnp.int32)
    mask_hi = jnp.full((16,), -65536, jnp.int32)

    def stage(t, b):
        n = roi0 + t // OUT
        ph = t % OUT
        pltpu.sync_copy(idxs.at[n, ph], idx_v[b])
        pltpu.sync_copy(ws.at[n, ph], w_v[b])
        pltpu.async_copy(table.at[idx_v[b]], rows_v[b], sems[b])

    stage(jnp.int32(0), 0)

    def tt_body(tt, _):
        for b in (0, 1):
            t = tt * 2 + b
            nb = 1 - b

            @pl.when(t + 1 < CHUNKS)
            def _():
                stage(t + 1, nb)

            pltpu.make_async_copy(table.at[idx_v[b]], rows_v[b], sems[b]).wait()
            n = roi0 + t // OUT
            ph = t % OUT

            def pw_body(pw, _, b=b):
                accs = [jnp.zeros((16,), jnp.float32) for _ in range(16)]
                for iy in range(2):
                    for j in range(8):
                        kk = iy * 56 + pw * 8 + j
                        wv = w_v[b][kk]
                        for c in range(8):
                            xi = rows_v[b][kk, pl.ds(c * 16, 16)]
                            lo = lax.bitcast_convert_type(
                                lax.shift_left(xi, shift16), jnp.float32
                            )
                            hi = lax.bitcast_convert_type(
                                lax.bitwise_and(xi, mask_hi), jnp.float32
                            )
                            accs[c] = accs[c] + wv * lo
                            accs[8 + c] = accs[8 + c] + wv * hi
                for c in range(8):
                    out_v[pw, pl.ds(c * 16, 16)] = accs[c]
                    out_v[pw, pl.ds(128 + c * 16, 16)] = accs[8 + c]
                return 0

            lax.fori_loop(0, OUT, pw_body, 0)
            pltpu.sync_copy(out_v, out.at[n, ph])
        return 0

    lax.fori_loop(0, CHUNKS // 2, tt_body, 0)


def _run_sc(table, idx, w):
    mesh = plsc.VectorSubcoreMesh(core_axis_name="c", subcore_axis_name="s")
    f = functools.partial(
        pl.kernel,
        out_type=jax.ShapeDtypeStruct((ROIS, OUT, OUT, C), jnp.float32),
        mesh=mesh,
        scratch_types=[
            pltpu.VMEM((112,), jnp.int32),
            pltpu.VMEM((112,), jnp.int32),
            pltpu.VMEM((112, 16), jnp.float32),
            pltpu.VMEM((112, 16), jnp.float32),
            pltpu.VMEM((112, CW), jnp.int32),
            pltpu.VMEM((112, CW), jnp.int32),
            pltpu.VMEM((OUT, C), jnp.float32),
            pltpu.SemaphoreType.DMA,
            pltpu.SemaphoreType.DMA,
        ],
    )(_sc_body)
    return f(table, idx, w)


_PERM = tuple(
    (p // 32) * 32 + (p % 2) * 16 + (p % 32) // 2 for p in range(C)
)


def kernel(feat_p2, feat_p3, feat_p4, feat_p5, boxes_raw):
    feats = (feat_p2, feat_p3, feat_p4, feat_p5)
    table_bf = jnp.concatenate(
        [
            f.transpose(0, 2, 3, 1).astype(jnp.bfloat16).reshape(-1, C)
            for f in feats
        ],
        axis=0,
    )
    u = lax.bitcast_convert_type(table_bf, jnp.uint16).astype(jnp.uint32)
    words = u[:, :CW] | (u[:, CW:] << 16)
    table = lax.bitcast_convert_type(words, jnp.int32)
    raw = boxes_raw.reshape(ROIS, 4)
    idx_flat, w_flat = _run_prologue(raw)
    idx = idx_flat.reshape(ROIS, OUT, 112)
    w = jnp.broadcast_to(
        w_flat.reshape(ROIS, OUT, 112, 1), (ROIS, OUT, 112, 16)
    )
    out = _run_sc(table, idx, w)
    return out.transpose(0, 3, 1, 2)


# speedup vs baseline: 2.9440x; 1.1154x over previous
import functools

import jax
import jax.numpy as jnp
from jax import lax
from jax.experimental import pallas as pl
from jax.experimental.pallas import tpu as pltpu, tpu_sc as plsc

OUT = 7
ROIS = 512
ROI_BLOCK = 64
C = 256
LVL_H = (200, 100, 50, 25)
LVL_BASE = (0, 80000, 100000, 105000)
LVL_BHW = (40000, 10000, 2500, 625)
TABLE_ROWS = 106250

NC, NS = 2, 16
NW = NC * NS
ROIS_PER_W = ROIS // NW


def _sel(lvl, vals, dtype):
    out = jnp.full(lvl.shape, vals[3], dtype=dtype)
    for l in (2, 1, 0):
        out = jnp.where(lvl == l, jnp.asarray(vals[l], dtype), out)
    return out


def _prologue_body(raw_ref, idx_ref, w_ref):
    raw = raw_ref[...]
    i = pl.program_id(0)
    rid = i * ROI_BLOCK + lax.broadcasted_iota(jnp.int32, (ROI_BLOCK, 1), 0)
    b = rid // 256

    x1 = raw[:, 0:1] * 600.0
    y1 = raw[:, 1:2] * 600.0
    x2 = x1 + 16.0 + raw[:, 2:3] * 200.0
    y2 = y1 + 16.0 + raw[:, 3:4] * 200.0
    area = (x2 - x1) * (y2 - y1)
    size = jnp.sqrt(area)
    lvlf = jnp.floor(4.0 + jnp.log2(size / 224.0 + 1e-8))
    lvl = jnp.clip(lvlf, 2.0, 5.0).astype(jnp.int32) - 2

    scale = _sel(lvl, (0.25, 0.125, 0.0625, 0.03125), jnp.float32)
    Hs = _sel(lvl, LVL_H, jnp.int32)
    Ws = Hs
    base = _sel(lvl, LVL_BASE, jnp.int32) + b * _sel(lvl, LVL_BHW, jnp.int32)
    Hf = Hs.astype(jnp.float32)
    Wf = Hf

    x1s = x1 * scale - 0.5
    y1s = y1 * scale - 0.5
    x2s = x2 * scale - 0.5
    y2s = y2 * scale - 0.5
    bin_w = (x2s - x1s) / OUT
    bin_h = (y2s - y1s) / OUT

    J = lax.broadcasted_iota(jnp.int32, (1, 784), 1)
    syi = J // 56
    sxi = (J % 56) // 4
    cr = J % 4
    gy = (syi // 2).astype(jnp.float32) + ((syi % 2).astype(jnp.float32) + 0.5) / 2.0
    gx = (sxi // 2).astype(jnp.float32) + ((sxi % 2).astype(jnp.float32) + 0.5) / 2.0

    sy = y1s + gy * bin_h
    sx = x1s + gx * bin_w
    vy = (sy > -1.0) & (sy < Hf)
    vx = (sx > -1.0) & (sx < Wf)
    yc = jnp.clip(sy, 0.0, Hf - 1.0)
    xc = jnp.clip(sx, 0.0, Wf - 1.0)
    y0f = jnp.floor(yc)
    x0f = jnp.floor(xc)
    y0i = y0f.astype(jnp.int32)
    x0i = x0f.astype(jnp.int32)
    y1i = jnp.minimum(y0i + 1, Hs - 1)
    x1i = jnp.minimum(x0i + 1, Ws - 1)
    ly = yc - y0f
    lx = xc - x0f
    hy = 1.0 - ly
    hx = 1.0 - lx

    yhi = cr >= 2
    xhi = (cr % 2) == 1
    yidx = jnp.where(yhi, y1i, y0i)
    wy = jnp.where(yhi, ly, hy)
    xidx = jnp.where(xhi, x1i, x0i)
    wx = jnp.where(xhi, lx, hx)

    idx = base + yidx * Ws + xidx
    wgt = jnp.where(vy & vx, wy * wx * 0.25, 0.0)
    idx_ref[...] = idx
    w_ref[...] = wgt


def _run_prologue(raw):
    grid = ROIS // ROI_BLOCK
    return pl.pallas_call(
        _prologue_body,
        grid=(grid,),
        in_specs=[pl.BlockSpec((ROI_BLOCK, 4), lambda i: (i, 0))],
        out_specs=[
            pl.BlockSpec((ROI_BLOCK, 784), lambda i: (i, 0)),
            pl.BlockSpec((ROI_BLOCK, 784), lambda i: (i, 0)),
        ],
        out_shape=[
            jax.ShapeDtypeStruct((ROIS, 784), jnp.int32),
            jax.ShapeDtypeStruct((ROIS, 784), jnp.float32),
        ],
    )(raw)


CHUNKS = ROIS_PER_W * OUT


def _sc_body(table, idxs, ws, out,
             idx0, idx1, w0, w1, rows0, rows1, out0, out1,
             sem0, sem1, osem0, osem1):
    idx_v = (idx0, idx1)
    w_v = (w0, w1)
    rows_v = (rows0, rows1)
    out_v = (out0, out1)
    sems = (sem0, sem1)
    osems = (osem0, osem1)
    wid = lax.axis_index("s") * NC + lax.axis_index("c")
    roi0 = wid * ROIS_PER_W

    def stage(t, b):
        n = roi0 + t // OUT
        ph = t % OUT
        pltpu.sync_copy(idxs.at[n, ph], idx_v[b])
        pltpu.sync_copy(ws.at[n, ph], w_v[b])
        pltpu.async_copy(table.at[idx_v[b]], rows_v[b], sems[b])

    stage(jnp.int32(0), 0)

    def tt_body(tt, _):
        for b in (0, 1):
            t = tt * 2 + b
            nb = 1 - b

            @pl.when(t + 1 < CHUNKS)
            def _():
                stage(t + 1, nb)

            pltpu.make_async_copy(table.at[idx_v[b]], rows_v[b], sems[b]).wait()
            n = roi0 + t // OUT
            ph = t % OUT

            @pl.when(t >= 2)
            def _():
                n2 = roi0 + (t - 2) // OUT
                ph2 = (t - 2) % OUT
                pltpu.make_async_copy(out_v[b], out.at[n2, ph2], osems[b]).wait()

            def pw_body(pw, _, b=b):
                accs = [jnp.zeros((16,), jnp.float32) for _ in range(16)]
                for iy in range(2):
                    for j in range(8):
                        kk = iy * 56 + pw * 8 + j
                        wv = w_v[b][kk]
                        for c in range(16):
                            accs[c] = accs[c] + wv * rows_v[b][kk, pl.ds(c * 16, 16)]
                for c in range(16):
                    out_v[b][pw, pl.ds(c * 16, 16)] = accs[c]
                return 0

            lax.fori_loop(0, OUT, pw_body, 0)
            pltpu.async_copy(out_v[b], out.at[n, ph], osems[b])
        return 0

    lax.fori_loop(0, CHUNKS // 2, tt_body, 0)

    for b in (0, 1):
        t = CHUNKS - 2 + b
        n2 = roi0 + t // OUT
        ph2 = t % OUT
        pltpu.make_async_copy(out_v[b], out.at[n2, ph2], osems[b]).wait()


def _run_sc(table, idx, w):
    mesh = plsc.VectorSubcoreMesh(core_axis_name="c", subcore_axis_name="s")
    f = functools.partial(
        pl.kernel,
        out_type=jax.ShapeDtypeStruct((ROIS, OUT, OUT, C), jnp.float32),
        mesh=mesh,
        scratch_types=[
            pltpu.VMEM((112,), jnp.int32),
            pltpu.VMEM((112,), jnp.int32),
            pltpu.VMEM((112, 16), jnp.float32),
            pltpu.VMEM((112, 16), jnp.float32),
            pltpu.VMEM((112, C), jnp.float32),
            pltpu.VMEM((112, C), jnp.float32),
            pltpu.VMEM((OUT, C), jnp.float32),
            pltpu.VMEM((OUT, C), jnp.float32),
            pltpu.SemaphoreType.DMA,
            pltpu.SemaphoreType.DMA,
            pltpu.SemaphoreType.DMA,
            pltpu.SemaphoreType.DMA,
        ],
    )(_sc_body)
    return f(table, idx, w)


def kernel(feat_p2, feat_p3, feat_p4, feat_p5, boxes_raw):
    feats = (feat_p2, feat_p3, feat_p4, feat_p5)
    table = jnp.concatenate(
        [f.transpose(0, 2, 3, 1).reshape(-1, C) for f in feats], axis=0
    )
    raw = boxes_raw.reshape(ROIS, 4)
    idx_flat, w_flat = _run_prologue(raw)
    idx = idx_flat.reshape(ROIS, OUT, 112)
    w = jnp.broadcast_to(
        w_flat.reshape(ROIS, OUT, 112, 1), (ROIS, OUT, 112, 16)
    )
    out = _run_sc(table, idx, w)
    return out.transpose(0, 3, 1, 2)
